# baseline (speedup 1.0000x reference)
import functools

import jax
import jax.numpy as jnp
from jax import lax
from jax.experimental import pallas as pl
from jax.experimental.pallas import tpu as pltpu
from jax.experimental.pallas import tpu_sc as plsc

N = 10000
E = 320000
IN_C = 128
H = 192
NG = 16
NW = 32
NP = 10240
E2 = E + N
C1 = 96
EPT = 10368
E2P = NW * EPT
C0 = 80
EPT0 = E // NW
NJ = H // 16
QH = H // 4
NJQ = QH // 16
TS = NP // 16
HP = 256
F32 = jnp.float32
I32 = jnp.int32


def _mesh():
    return plsc.VectorSubcoreMesh(core_axis_name="c", subcore_axis_name="s",
                                  num_cores=2, num_subcores=16)


_SC_PARAMS = pltpu.CompilerParams(needs_layout_passes=False,
                                  use_tc_tiling_on_sc=False)


def _emb_body(x_ref, we_ref, be_ref, wl_ref, bl_ref, wr_ref, br_ref,
              emb_ref, s_ref, xr_ref, xl_ref, q0_ref, q1_ref, q2_ref,
              q3_ref):
    emb = jnp.dot(x_ref[...], we_ref[...],
                  preferred_element_type=F32) + be_ref[...]
    s = emb * jax.nn.sigmoid(emb)
    emb_ref[...] = emb
    s_ref[...] = s
    xl = jnp.dot(s, wl_ref[...], preferred_element_type=F32) + bl_ref[...]
    xl_ref[...] = xl
    q0_ref[...] = xl[:, 0 * QH:1 * QH]
    q1_ref[...] = xl[:, 1 * QH:2 * QH]
    q2_ref[...] = xl[:, 2 * QH:3 * QH]
    q3_ref[...] = xl[:, 3 * QH:4 * QH]
    xr_ref[...] = jnp.dot(s, wr_ref[...],
                          preferred_element_type=F32) + br_ref[...]


def _tc_embed(x, W_emb, b_emb, Wl, bl, Wr, br):
    R = 1000
    full = lambda i: (0, 0)
    return pl.pallas_call(
        _emb_body,
        grid=(N // R,),
        in_specs=[
            pl.BlockSpec((R, IN_C), lambda i: (i, 0)),
            pl.BlockSpec((IN_C, H), full),
            pl.BlockSpec((1, H), full),
            pl.BlockSpec((H, H), full),
            pl.BlockSpec((1, H), full),
            pl.BlockSpec((H, H), full),
            pl.BlockSpec((1, H), full),
        ],
        out_specs=[pl.BlockSpec((R, H), lambda i: (i, 0))] * 4
        + [pl.BlockSpec((R, QH), lambda i: (i, 0))] * 4,
        out_shape=[jax.ShapeDtypeStruct((N, H), F32)] * 4
        + [jax.ShapeDtypeStruct((N, QH), F32)] * 4,
    )(x, W_emb, b_emb.reshape(1, H), Wl, bl.reshape(1, H), Wr,
      br.reshape(1, H))


def _gmm_body(ea_ref, sc_ref, wb_ref, g_ref):
    g_ref[...] = jnp.dot(ea_ref[...] * sc_ref[...], wb_ref[...],
                         preferred_element_type=F32)


def _tc_edge_embed(ea2p8, scale8, W_big):
    BLK8 = 256
    E8 = E2P // 8
    return pl.pallas_call(
        _gmm_body,
        grid=(E8 // BLK8,),
        in_specs=[
            pl.BlockSpec((BLK8, 128), lambda i: (i, 0)),
            pl.BlockSpec((BLK8, 128), lambda i: (i, 0)),
            pl.BlockSpec((128, 8 * H), lambda i: (0, 0)),
        ],
        out_specs=pl.BlockSpec((BLK8, 8 * H), lambda i: (i, 0)),
        out_shape=jax.ShapeDtypeStruct((E8, 8 * H), F32),
    )(ea2p8, scale8, W_big)


def _fin_body(o00, o01, o10, o11, o20, o21, o30, o31, d0_ref, d1_ref,
              s_ref, bias_ref, out_ref):
    inv = 1.0 / (d0_ref[0] + d1_ref[0])
    qs = []
    for q, (a, b) in enumerate(((o00, o01), (o10, o11), (o20, o21),
                                (o30, o31))):
        qs.append((a[0, 0] + b[0, 0]) * inv
                  + bias_ref[0, q * QH:(q + 1) * QH])
    t = jnp.concatenate(qs, axis=-1)
    out_ref[...] = s_ref[...] + t * jax.nn.sigmoid(t)


def _tc_finish(outq, den_p, s, bias):
    R = 1000
    qspecs = []
    for q in range(4):
        for c in range(2):
            qspecs.append(pl.BlockSpec(
                (1, 1, R, QH),
                functools.partial(lambda i, _q, _c: (_q, _c, i, 0),
                                  _q=q, _c=c)))
    return pl.pallas_call(
        _fin_body,
        grid=(N // R,),
        in_specs=qspecs + [
            pl.BlockSpec((1, R, 1), lambda i: (0, i, 0)),
            pl.BlockSpec((1, R, 1), lambda i: (1, i, 0)),
            pl.BlockSpec((R, H), lambda i: (i, 0)),
            pl.BlockSpec((1, H), lambda i: (0, 0)),
        ],
        out_specs=pl.BlockSpec((R, H), lambda i: (i, 0)),
        out_shape=jax.ShapeDtypeStruct((N, H), F32),
    )(*([outq] * 8), den_p.reshape(2, NP, 1), den_p.reshape(2, NP, 1),
      s, bias.reshape(1, H))


SB0 = 5
NSC0 = EPT0 // (SB0 * C0)


def _sc_edge_attr(posp, rowm, colm):
    @functools.partial(
        pl.kernel,
        out_type=[
            jax.ShapeDtypeStruct((E * NG,), F32),
            jax.ShapeDtypeStruct((2, NP, NG), F32),
            jax.ShapeDtypeStruct((2, NP), F32),
        ],
        mesh=_mesh(),
        compiler_params=_SC_PARAMS,
        scratch_types=[
            pltpu.VMEM((SB0, C0), I32),
            pltpu.VMEM((SB0, C0), I32),
            pltpu.VMEM((C0, 16), F32),
            pltpu.VMEM((C0, 16), F32),
            pltpu.VMEM((C0, NG), F32),
            pltpu.VMEM((C0 * NG,), F32),
            pltpu.VMEM((C0,), F32),
            pltpu.VMEM((TS,), F32),
            pltpu.VMEM_SHARED((NP, NG), F32),
            pltpu.VMEM_SHARED((NP,), F32),
            pltpu.SemaphoreType.DMA,
            pltpu.SemaphoreType.DMA,
            pltpu.SemaphoreType.DMA,
        ],
    )
    def k(pos_hbm, rowm_hbm, colm_hbm, ea_hbm, lea_hbm, deg_hbm,
          rw0, cw0, pr, pc, eava, eapk, onesv, zb1, lea_sh, deg_sh,
          semg, semoa, semob):
        cid = lax.axis_index("c")
        sid = lax.axis_index("s")
        wid = sid * 2 + cid
        zv16 = jnp.zeros((16,), F32)
        one16 = jnp.full((16,), 1.0, F32)

        def zea(e, carry):
            eava[e, :] = zv16
            return carry
        lax.fori_loop(0, C0, zea, 0)

        def zf1(i, carry):
            zb1[pl.ds(i * 16, 16)] = zv16
            return carry
        lax.fori_loop(0, TS // 16, zf1, 0)

        def of(i, carry):
            onesv[pl.ds(i * 16, 16)] = one16
            return carry
        lax.fori_loop(0, C0 // 16, of, 0)

        for kk in range(TS // C0):
            pltpu.sync_copy(eava,
                            lea_sh.at[pl.ds(sid * TS + kk * C0, C0)])
        pltpu.sync_copy(zb1, deg_sh.at[pl.ds(sid * TS, TS)])
        plsc.subcore_barrier()

        offs = (lax.iota(I32, 16).astype(F32) * jnp.float32(10.0 / 15.0))
        nch = EPT0 // C0

        def schunk(g, carry):
            sbase = wid * nch + g * SB0
            pltpu.sync_copy(rowm_hbm.at[pl.ds(sbase, SB0)], rw0)
            pltpu.sync_copy(colm_hbm.at[pl.ds(sbase, SB0)], cw0)
            for j in range(SB0):
                dr = pltpu.async_copy(pos_hbm.at[rw0.at[j]], pr, semg)
                dc = pltpu.async_copy(pos_hbm.at[cw0.at[j]], pc, semg)
                dr.wait()
                dc.wait()
                eav = eava

                def edge(e, c2_, _eav=eav):
                    dv = pr[e, :] - pc[e, :]
                    d2 = jnp.sum(dv * dv) + jnp.float32(1e-12)
                    d2v = jnp.full((16,), d2, F32)
                    ii = plsc.bitcast(d2v, I32)
                    ii = (jnp.int32(0x5F3759DF)
                          - lax.shift_right_logical(ii, 1))
                    y = plsc.bitcast(ii, F32)
                    for _ in range(4):
                        y = y * (jnp.float32(1.5)
                                 - jnp.float32(0.5) * d2v * y * y)
                    tt = d2v * y - offs
                    ea = jnp.exp(jnp.float32(-1.125) * tt * tt)
                    _eav[e, :] = ea
                    eapk[pl.ds(e * NG, NG)] = ea
                    return c2_
                lax.fori_loop(0, C0, edge, 0)

                basef = (sbase + j) * (C0 * NG)
                pltpu.sync_copy(eapk, ea_hbm.at[pl.ds(basef, C0 * NG)])
                pltpu.sync_copy(eav, lea_sh.at[cw0.at[j]], add=True)
                pltpu.sync_copy(onesv, deg_sh.at[cw0.at[j]], add=True)
            return carry
        lax.fori_loop(0, NSC0, schunk, 0)
        plsc.subcore_barrier()
        pltpu.sync_copy(lea_sh.at[pl.ds(sid * TS, TS)],
                        lea_hbm.at[cid].at[pl.ds(sid * TS, TS)])
        pltpu.sync_copy(deg_sh.at[pl.ds(sid * TS, TS)],
                        deg_hbm.at[cid].at[pl.ds(sid * TS, TS)])

    return k(posp, rowm, colm)


SB = 9
CB = SB * C1
NSC = EPT // CB


def _sc_pass1(xl, xr, g2, row2m, col2m, att):
    @functools.partial(
        pl.kernel,
        out_type=[
            jax.ShapeDtypeStruct((E2P,), F32),
            jax.ShapeDtypeStruct((2, NP), F32),
        ],
        mesh=_mesh(),
        compiler_params=_SC_PARAMS,
        scratch_types=[
            pltpu.VMEM((SB, C1), I32),
            pltpu.VMEM((SB, C1), I32),
            pltpu.VMEM((C1, H), F32),
            pltpu.VMEM((C1, H), F32),
            pltpu.VMEM((C1 // 8, 8 * H), F32),
            pltpu.VMEM((CB,), F32),
            pltpu.VMEM((H,), F32),
            pltpu.VMEM((TS,), F32),
            pltpu.VMEM_SHARED((NP,), F32),
            pltpu.SemaphoreType.DMA,
            pltpu.SemaphoreType.DMA,
        ],
    )
    def k(xl_hbm, xr_hbm, g2_hbm, row2m_hbm, col2m_hbm, att_hbm,
          z_hbm, den_hbm,
          rw2, cw2, abuf, bbuf, gbuf, zwide, attv, zb1, den_sh,
          semg, semd):
        cid = lax.axis_index("c")
        sid = lax.axis_index("s")
        wid = sid * 2 + cid
        zv16 = jnp.zeros((16,), F32)

        def zf1(i, carry):
            zb1[pl.ds(i * 16, 16)] = zv16
            return carry
        lax.fori_loop(0, TS // 16, zf1, 0)
        pltpu.sync_copy(zb1, den_sh.at[pl.ds(sid * TS, TS)])
        pltpu.sync_copy(att_hbm, attv)
        plsc.subcore_barrier()

        lane = lax.iota(I32, 16)

        def schunk(g, carry):
            sbase = wid * (EPT // C1) + g * SB
            ebase = sbase * C1
            pltpu.sync_copy(row2m_hbm.at[pl.ds(sbase, SB)], rw2)
            pltpu.sync_copy(col2m_hbm.at[pl.ds(sbase, SB)], cw2)
            dscat = []
            for j in range(SB):
                da = pltpu.async_copy(xl_hbm.at[rw2.at[j]], abuf, semg)
                db = pltpu.async_copy(xr_hbm.at[cw2.at[j]], bbuf, semg)
                dg = pltpu.async_copy(
                    g2_hbm.at[pl.ds((ebase + j * C1) // 8, C1 // 8)],
                    gbuf, semg)
                da.wait()
                db.wait()
                dg.wait()

                def grp16(i, carry2):
                    vec = jnp.zeros((16,), F32)
                    for p in range(16):
                        e = i * 16 + p
                        acc = jnp.zeros((16,), F32)
                        e8 = lax.shift_right_logical(e, 3)
                        eo = (e & 7) * H
                        for jj in range(NJ):
                            sl = pl.ds(jj * 16, 16)
                            tt = (abuf[e, sl] + bbuf[e, sl]
                                  + gbuf[e8, pl.ds(eo + jj * 16, 16)])
                            lk = (jnp.maximum(tt, 0.0)
                                  + jnp.float32(0.2)
                                  * jnp.minimum(tt, 0.0))
                            acc = acc + lk * attv[sl]
                        lg = jnp.sum(acc)
                        vec = jnp.where(lane == p,
                                        jnp.full((16,), lg, F32), vec)
                    gidx = (jnp.full((16,), ebase + j * C1 + i * 16, I32)
                            + lane)
                    zz = jnp.exp(vec)
                    zwide[pl.ds(j * C1 + i * 16, 16)] = (
                        jnp.where(gidx < E2, zz, 0.0))
                    return carry2
                lax.fori_loop(0, C1 // 16, grp16, 0)

                dscat.append(pltpu.async_copy(
                    zwide.at[pl.ds(j * C1, C1)],
                    den_sh.at[cw2.at[j]], semd, add=True))

            pltpu.sync_copy(zwide, z_hbm.at[pl.ds(ebase, CB)])
            for d in dscat:
                d.wait()
            return carry
        lax.fori_loop(0, NSC, schunk, 0)
        plsc.subcore_barrier()
        pltpu.sync_copy(den_sh.at[pl.ds(sid * TS, TS)],
                        den_hbm.at[cid].at[pl.ds(sid * TS, TS)])

    return k(xl, xr, g2, row2m, col2m, att)


def _sc_pass2(xlq, z, row2m, col2m):
    @functools.partial(
        pl.kernel,
        out_type=jax.ShapeDtypeStruct((4, 2, NP, QH), F32),
        mesh=_mesh(),
        compiler_params=_SC_PARAMS,
        scratch_types=[
            pltpu.VMEM((SB, C1), I32),
            pltpu.VMEM((SB, C1), I32),
            pltpu.VMEM((CB + 16,), F32),
            pltpu.VMEM((C1, QH), F32),
            pltpu.VMEM((C1, QH), F32),
            pltpu.VMEM_SHARED((NP, QH), F32),
            pltpu.SemaphoreType.DMA,
            pltpu.SemaphoreType.DMA,
            pltpu.SemaphoreType.DMA,
            pltpu.SemaphoreType.DMA,
        ],
    )
    def k(q0_hbm, q1_hbm, q2_hbm, q3_hbm, z_hbm, row2m_hbm, col2m_hbm,
          out_hbm,
          rw2, cw2, zwide, abufa, abufb, out_sh,
          semga, semgb, semsa, semsb):
        cid = lax.axis_index("c")
        sid = lax.axis_index("s")
        wid = sid * 2 + cid
        zv16 = jnp.zeros((16,), F32)
        bufs = (abufa, abufb)
        semg = (semga, semgb)
        sems = (semsa, semsb)
        q_hbm = (q0_hbm, q1_hbm, q2_hbm, q3_hbm)

        def zfa(e, carry):
            for j in range(NJQ):
                abufa[e, pl.ds(j * 16, 16)] = zv16
            return carry
        lax.fori_loop(0, C1, zfa, 0)

        for h in range(4):
            for kk in range(TS // C1):
                pltpu.sync_copy(abufa,
                                out_sh.at[pl.ds(sid * TS + kk * C1, C1)])
            rem = TS - (TS // C1) * C1
            if rem:
                pltpu.sync_copy(
                    abufa.at[pl.ds(0, rem)],
                    out_sh.at[pl.ds(sid * TS + (TS // C1) * C1, rem)])
            plsc.subcore_barrier()

            def schunk(g, carry):
                sbase = wid * (EPT // C1) + g * SB
                ebase = sbase * C1
                pltpu.sync_copy(row2m_hbm.at[pl.ds(sbase, SB)], rw2)
                pltpu.sync_copy(col2m_hbm.at[pl.ds(sbase, SB)], cw2)
                pltpu.sync_copy(z_hbm.at[pl.ds(ebase, CB)],
                                zwide.at[pl.ds(0, CB)])
                gd = {0: pltpu.async_copy(q_hbm[h].at[rw2.at[0]],
                                          bufs[0], semg[0])}

                scat = [None, None]
                for j in range(SB):
                    b = j % 2
                    gd[j].wait()
                    if j + 1 < SB:
                        ob = (j + 1) % 2
                        if scat[ob] is not None:
                            scat[ob].wait()
                            scat[ob] = None
                        gd[j + 1] = pltpu.async_copy(
                            q_hbm[h].at[rw2.at[j + 1]], bufs[ob],
                            semg[ob])

                    buf = bufs[b]

                    def edge(e, carry2, _j=j, _buf=buf):
                        asc = zwide[pl.ds(_j * C1 + e, 16)][0]
                        av = jnp.full((16,), asc, F32)
                        for jj in range(NJQ):
                            sl = pl.ds(jj * 16, 16)
                            _buf[e, sl] = _buf[e, sl] * av
                        return carry2
                    lax.fori_loop(0, C1, edge, 0)

                    scat[b] = pltpu.async_copy(
                        buf, out_sh.at[cw2.at[j]], sems[b], add=True)
                for b in range(2):
                    if scat[b] is not None:
                        scat[b].wait()
                return carry
            lax.fori_loop(0, NSC, schunk, 0)
            plsc.subcore_barrier()
            pltpu.sync_copy(out_sh.at[pl.ds(sid * TS, TS)],
                            out_hbm.at[h].at[cid].at[pl.ds(sid * TS, TS)])
            plsc.subcore_barrier()
            lax.fori_loop(0, C1, zfa, 0)

    return k(*xlq, z, row2m, col2m)


def kernel(x, edge_index, pos, W_emb, b_emb, Wl, bl, Wr, br, We, att, bias):
    row = edge_index[0]
    col = edge_index[1]
    posp = jnp.pad(pos, ((0, 0), (0, 13)))

    emb, s, xr, xl, q0, q1, q2, q3 = _tc_embed(x, W_emb, b_emb, Wl, bl,
                                               Wr, br)
    xlq = (q0, q1, q2, q3)

    eaE, lea_p, deg_p = _sc_edge_attr(posp, row.reshape(E // C0, C0),
                                      col.reshape(E // C0, C0))
    lea = lea_p[0, :N] + lea_p[1, :N]
    deg = deg_p[0, :N] + deg_p[1, :N]

    pad_i = jnp.zeros((E2P - E2,), I32)
    ar = jnp.arange(N, dtype=I32)
    row2 = jnp.concatenate([row, ar, pad_i])
    col2 = jnp.concatenate([col, ar, pad_i])
    ea2p8 = jnp.concatenate(
        [eaE.reshape(E * NG // 128, 128), lea.reshape(N * NG // 128, 128),
         jnp.zeros(((E2P - E2) // 8, 128), F32)], axis=0)
    inv = 1.0 / jnp.maximum(
        jnp.concatenate([jnp.ones((E,), F32), deg,
                         jnp.ones((E2P - E2,), F32)]), 1.0)
    scale8 = jnp.repeat(inv, NG).reshape(E2P // 8, 128)
    W_big = jnp.zeros((128, 8 * H), F32)
    for kq in range(8):
        W_big = lax.dynamic_update_slice(W_big, We, (kq * NG, kq * H))

    g2 = _tc_edge_embed(ea2p8, scale8, W_big)

    row2m = row2.reshape(E2P // C1, C1)
    col2m = col2.reshape(E2P // C1, C1)
    z, den_p = _sc_pass1(xl, xr, g2, row2m, col2m, att)
    outq = _sc_pass2(xlq, z, row2m, col2m)

    s_out = _tc_finish(outq, den_p, s, bias)
    v = emb[:, :48].reshape(N, 16, 3)
    return (s_out, v)


# speedup vs baseline: 1.0279x; 1.0279x over previous
import functools

import jax
import jax.numpy as jnp
from jax import lax
from jax.experimental import pallas as pl
from jax.experimental.pallas import tpu as pltpu
from jax.experimental.pallas import tpu_sc as plsc

N = 10000
E = 320000
IN_C = 128
H = 192
NG = 16
NW = 32
NP = 10240
E2 = E + N
C1 = 96
EPT = 10368
E2P = NW * EPT
C0 = 80
EPT0 = E // NW
NJ = H // 16
QH = H // 4
NJQ = QH // 16
TS = NP // 16
HP = 256
F32 = jnp.float32
I32 = jnp.int32


def _mesh():
    return plsc.VectorSubcoreMesh(core_axis_name="c", subcore_axis_name="s",
                                  num_cores=2, num_subcores=16)


_SC_PARAMS = pltpu.CompilerParams(needs_layout_passes=False,
                                  use_tc_tiling_on_sc=False)


def _emb_body(x_ref, we_ref, be_ref, wl_ref, bl_ref, wr_ref, br_ref,
              emb_ref, s_ref, xr_ref, xl_ref, q0_ref, q1_ref, q2_ref,
              q3_ref):
    emb = jnp.dot(x_ref[...], we_ref[...],
                  preferred_element_type=F32) + be_ref[...]
    s = emb * jax.nn.sigmoid(emb)
    emb_ref[...] = emb
    s_ref[...] = s
    xl = jnp.dot(s, wl_ref[...], preferred_element_type=F32) + bl_ref[...]
    xl_ref[...] = xl
    q0_ref[...] = xl[:, 0 * QH:1 * QH]
    q1_ref[...] = xl[:, 1 * QH:2 * QH]
    q2_ref[...] = xl[:, 2 * QH:3 * QH]
    q3_ref[...] = xl[:, 3 * QH:4 * QH]
    xr_ref[...] = jnp.dot(s, wr_ref[...],
                          preferred_element_type=F32) + br_ref[...]


def _tc_embed(x, W_emb, b_emb, Wl, bl, Wr, br):
    R = 1000
    full = lambda i: (0, 0)
    return pl.pallas_call(
        _emb_body,
        grid=(N // R,),
        in_specs=[
            pl.BlockSpec((R, IN_C), lambda i: (i, 0)),
            pl.BlockSpec((IN_C, H), full),
            pl.BlockSpec((1, H), full),
            pl.BlockSpec((H, H), full),
            pl.BlockSpec((1, H), full),
            pl.BlockSpec((H, H), full),
            pl.BlockSpec((1, H), full),
        ],
        out_specs=[pl.BlockSpec((R, H), lambda i: (i, 0))] * 4
        + [pl.BlockSpec((R, QH), lambda i: (i, 0))] * 4,
        out_shape=[jax.ShapeDtypeStruct((N, H), F32)] * 4
        + [jax.ShapeDtypeStruct((N, QH), F32)] * 4,
    )(x, W_emb, b_emb.reshape(1, H), Wl, bl.reshape(1, H), Wr,
      br.reshape(1, H))


def _gmm_body(ea_ref, sc_ref, wb_ref, g_ref):
    g_ref[...] = jnp.dot(ea_ref[...] * sc_ref[...], wb_ref[...],
                         preferred_element_type=F32)


def _tc_edge_embed(ea2p8, scale8, W_big):
    BLK8 = 256
    E8 = E2P // 8
    return pl.pallas_call(
        _gmm_body,
        grid=(E8 // BLK8,),
        in_specs=[
            pl.BlockSpec((BLK8, 128), lambda i: (i, 0)),
            pl.BlockSpec((BLK8, 128), lambda i: (i, 0)),
            pl.BlockSpec((128, 8 * H), lambda i: (0, 0)),
        ],
        out_specs=pl.BlockSpec((BLK8, 8 * H), lambda i: (i, 0)),
        out_shape=jax.ShapeDtypeStruct((E8, 8 * H), F32),
    )(ea2p8, scale8, W_big)


def _fin_body(o00, o01, o10, o11, o20, o21, o30, o31, d0_ref, d1_ref,
              s_ref, bias_ref, out_ref):
    inv = 1.0 / (d0_ref[0] + d1_ref[0])
    qs = []
    for q, (a, b) in enumerate(((o00, o01), (o10, o11), (o20, o21),
                                (o30, o31))):
        qs.append((a[0, 0] + b[0, 0]) * inv
                  + bias_ref[0, q * QH:(q + 1) * QH])
    t = jnp.concatenate(qs, axis=-1)
    out_ref[...] = s_ref[...] + t * jax.nn.sigmoid(t)


def _tc_finish(outq, den_p, s, bias):
    R = 1000
    qspecs = []
    for q in range(4):
        for c in range(2):
            qspecs.append(pl.BlockSpec(
                (1, 1, R, QH),
                functools.partial(lambda i, _q, _c: (_q, _c, i, 0),
                                  _q=q, _c=c)))
    return pl.pallas_call(
        _fin_body,
        grid=(N // R,),
        in_specs=qspecs + [
            pl.BlockSpec((1, R, 1), lambda i: (0, i, 0)),
            pl.BlockSpec((1, R, 1), lambda i: (1, i, 0)),
            pl.BlockSpec((R, H), lambda i: (i, 0)),
            pl.BlockSpec((1, H), lambda i: (0, 0)),
        ],
        out_specs=pl.BlockSpec((R, H), lambda i: (i, 0)),
        out_shape=jax.ShapeDtypeStruct((N, H), F32),
    )(*([outq] * 8), den_p.reshape(2, NP, 1), den_p.reshape(2, NP, 1),
      s, bias.reshape(1, H))


SB0 = 5
NSC0 = EPT0 // (SB0 * C0)


def _sc_edge_attr(posp, rowm, colm):
    @functools.partial(
        pl.kernel,
        out_type=[
            jax.ShapeDtypeStruct((E * NG,), F32),
            jax.ShapeDtypeStruct((2, NP, NG), F32),
            jax.ShapeDtypeStruct((2, NP), F32),
        ],
        mesh=_mesh(),
        compiler_params=_SC_PARAMS,
        scratch_types=[
            pltpu.VMEM((SB0, C0), I32),
            pltpu.VMEM((SB0, C0), I32),
            pltpu.VMEM((C0, 16), F32),
            pltpu.VMEM((C0, 16), F32),
            pltpu.VMEM((C0, NG), F32),
            pltpu.VMEM((C0 * NG,), F32),
            pltpu.VMEM((C0,), F32),
            pltpu.VMEM((TS,), F32),
            pltpu.VMEM_SHARED((NP, NG), F32),
            pltpu.VMEM_SHARED((NP,), F32),
            pltpu.SemaphoreType.DMA,
            pltpu.SemaphoreType.DMA,
            pltpu.SemaphoreType.DMA,
        ],
    )
    def k(pos_hbm, rowm_hbm, colm_hbm, ea_hbm, lea_hbm, deg_hbm,
          rw0, cw0, pr, pc, eava, eapk, onesv, zb1, lea_sh, deg_sh,
          semg, semoa, semob):
        cid = lax.axis_index("c")
        sid = lax.axis_index("s")
        wid = sid * 2 + cid
        zv16 = jnp.zeros((16,), F32)
        one16 = jnp.full((16,), 1.0, F32)

        def zea(e, carry):
            eava[e, :] = zv16
            return carry
        lax.fori_loop(0, C0, zea, 0)

        def zf1(i, carry):
            zb1[pl.ds(i * 16, 16)] = zv16
            return carry
        lax.fori_loop(0, TS // 16, zf1, 0)

        def of(i, carry):
            onesv[pl.ds(i * 16, 16)] = one16
            return carry
        lax.fori_loop(0, C0 // 16, of, 0)

        for kk in range(TS // C0):
            pltpu.sync_copy(eava,
                            lea_sh.at[pl.ds(sid * TS + kk * C0, C0)])
        pltpu.sync_copy(zb1, deg_sh.at[pl.ds(sid * TS, TS)])
        plsc.subcore_barrier()

        offs = (lax.iota(I32, 16).astype(F32) * jnp.float32(10.0 / 15.0))
        nch = EPT0 // C0

        def schunk(g, carry):
            sbase = wid * nch + g * SB0
            pltpu.sync_copy(rowm_hbm.at[pl.ds(sbase, SB0)], rw0)
            pltpu.sync_copy(colm_hbm.at[pl.ds(sbase, SB0)], cw0)
            for j in range(SB0):
                dr = pltpu.async_copy(pos_hbm.at[rw0.at[j]], pr, semg)
                dc = pltpu.async_copy(pos_hbm.at[cw0.at[j]], pc, semg)
                dr.wait()
                dc.wait()
                eav = eava

                def edge(e, c2_, _eav=eav):
                    dv = pr[e, :] - pc[e, :]
                    d2 = jnp.sum(dv * dv) + jnp.float32(1e-12)
                    d2v = jnp.full((16,), d2, F32)
                    ii = plsc.bitcast(d2v, I32)
                    ii = (jnp.int32(0x5F3759DF)
                          - lax.shift_right_logical(ii, 1))
                    y = plsc.bitcast(ii, F32)
                    for _ in range(4):
                        y = y * (jnp.float32(1.5)
                                 - jnp.float32(0.5) * d2v * y * y)
                    tt = d2v * y - offs
                    ea = jnp.exp(jnp.float32(-1.125) * tt * tt)
                    _eav[e, :] = ea
                    eapk[pl.ds(e * NG, NG)] = ea
                    return c2_
                lax.fori_loop(0, C0, edge, 0)

                basef = (sbase + j) * (C0 * NG)
                pltpu.sync_copy(eapk, ea_hbm.at[pl.ds(basef, C0 * NG)])
                pltpu.sync_copy(eav, lea_sh.at[cw0.at[j]], add=True)
                pltpu.sync_copy(onesv, deg_sh.at[cw0.at[j]], add=True)
            return carry
        lax.fori_loop(0, NSC0, schunk, 0)
        plsc.subcore_barrier()
        pltpu.sync_copy(lea_sh.at[pl.ds(sid * TS, TS)],
                        lea_hbm.at[cid].at[pl.ds(sid * TS, TS)])
        pltpu.sync_copy(deg_sh.at[pl.ds(sid * TS, TS)],
                        deg_hbm.at[cid].at[pl.ds(sid * TS, TS)])

    return k(posp, rowm, colm)


SB = 9
CB = SB * C1
NSC = EPT // CB


def _sc_pass1(xl, xr, g2, row2m, col2m, att):
    @functools.partial(
        pl.kernel,
        out_type=[
            jax.ShapeDtypeStruct((E2P,), F32),
            jax.ShapeDtypeStruct((2, NP), F32),
        ],
        mesh=_mesh(),
        compiler_params=_SC_PARAMS,
        scratch_types=[
            pltpu.VMEM((SB, C1), I32),
            pltpu.VMEM((SB, C1), I32),
            pltpu.VMEM((C1, H), F32),
            pltpu.VMEM((C1, H), F32),
            pltpu.VMEM((C1 // 8, 8 * H), F32),
            pltpu.VMEM((CB,), F32),
            pltpu.VMEM((H,), F32),
            pltpu.VMEM((TS,), F32),
            pltpu.VMEM_SHARED((NP,), F32),
            pltpu.SemaphoreType.DMA,
            pltpu.SemaphoreType.DMA,
        ],
    )
    def k(xl_hbm, xr_hbm, g2_hbm, row2m_hbm, col2m_hbm, att_hbm,
          z_hbm, den_hbm,
          rw2, cw2, abuf, bbuf, gbuf, zwide, attv, zb1, den_sh,
          semg, semd):
        cid = lax.axis_index("c")
        sid = lax.axis_index("s")
        wid = sid * 2 + cid
        zv16 = jnp.zeros((16,), F32)

        def zf1(i, carry):
            zb1[pl.ds(i * 16, 16)] = zv16
            return carry
        lax.fori_loop(0, TS // 16, zf1, 0)
        pltpu.sync_copy(zb1, den_sh.at[pl.ds(sid * TS, TS)])
        pltpu.sync_copy(att_hbm, attv)
        plsc.subcore_barrier()

        lane = lax.iota(I32, 16)

        def schunk(g, carry):
            sbase = wid * (EPT // C1) + g * SB
            ebase = sbase * C1
            pltpu.sync_copy(row2m_hbm.at[pl.ds(sbase, SB)], rw2)
            pltpu.sync_copy(col2m_hbm.at[pl.ds(sbase, SB)], cw2)
            dscat = []
            for j in range(SB):
                da = pltpu.async_copy(xl_hbm.at[rw2.at[j]], abuf, semg)
                db = pltpu.async_copy(xr_hbm.at[cw2.at[j]], bbuf, semg)
                dg = pltpu.async_copy(
                    g2_hbm.at[pl.ds((ebase + j * C1) // 8, C1 // 8)],
                    gbuf, semg)
                da.wait()
                db.wait()
                dg.wait()

                def grp16(i, carry2):
                    def edge(p, vec):
                        e = i * 16 + p
                        acc = jnp.zeros((16,), F32)
                        e8 = lax.shift_right_logical(e, 3)
                        eo = (e & 7) * H
                        for jj in range(NJ):
                            sl = pl.ds(jj * 16, 16)
                            tt = (abuf[e, sl] + bbuf[e, sl]
                                  + gbuf[e8, pl.ds(eo + jj * 16, 16)])
                            lk = (jnp.maximum(tt, 0.0)
                                  + jnp.float32(0.2)
                                  * jnp.minimum(tt, 0.0))
                            acc = acc + lk * attv[sl]
                        lg = jnp.sum(acc)
                        return jnp.where(lane == p,
                                         jnp.full((16,), lg, F32), vec)
                    vec = lax.fori_loop(0, 16, edge,
                                        jnp.zeros((16,), F32))
                    gidx = (jnp.full((16,), ebase + j * C1 + i * 16, I32)
                            + lane)
                    zz = jnp.exp(vec)
                    zwide[pl.ds(j * C1 + i * 16, 16)] = (
                        jnp.where(gidx < E2, zz, 0.0))
                    return carry2
                lax.fori_loop(0, C1 // 16, grp16, 0)

                dscat.append(pltpu.async_copy(
                    zwide.at[pl.ds(j * C1, C1)],
                    den_sh.at[cw2.at[j]], semd, add=True))

            pltpu.sync_copy(zwide, z_hbm.at[pl.ds(ebase, CB)])
            for d in dscat:
                d.wait()
            return carry
        lax.fori_loop(0, NSC, schunk, 0)
        plsc.subcore_barrier()
        pltpu.sync_copy(den_sh.at[pl.ds(sid * TS, TS)],
                        den_hbm.at[cid].at[pl.ds(sid * TS, TS)])

    return k(xl, xr, g2, row2m, col2m, att)


C2 = 128
SB2 = 9
CB2 = SB2 * C2
NSC2 = EPT // CB2


def _sc_pass2(xlq, z, row2m, col2m):
    @functools.partial(
        pl.kernel,
        out_type=jax.ShapeDtypeStruct((4, 2, NP, QH), F32),
        mesh=_mesh(),
        compiler_params=_SC_PARAMS,
        scratch_types=[
            pltpu.VMEM((SB2, C2), I32),
            pltpu.VMEM((SB2, C2), I32),
            pltpu.VMEM((CB2 + 16,), F32),
            pltpu.VMEM((C2, QH), F32),
            pltpu.VMEM((C2, QH), F32),
            pltpu.VMEM_SHARED((NP, QH), F32),
            pltpu.SemaphoreType.DMA,
            pltpu.SemaphoreType.DMA,
            pltpu.SemaphoreType.DMA,
            pltpu.SemaphoreType.DMA,
        ],
    )
    def k(q0_hbm, q1_hbm, q2_hbm, q3_hbm, z_hbm, row2m_hbm, col2m_hbm,
          out_hbm,
          rw2, cw2, zwide, abufa, abufb, out_sh,
          semga, semgb, semsa, semsb):
        cid = lax.axis_index("c")
        sid = lax.axis_index("s")
        wid = sid * 2 + cid
        zv16 = jnp.zeros((16,), F32)
        bufs = (abufa, abufb)
        semg = (semga, semgb)
        sems = (semsa, semsb)
        q_hbm = (q0_hbm, q1_hbm, q2_hbm, q3_hbm)

        def zfa(e, carry):
            for j in range(NJQ):
                abufa[e, pl.ds(j * 16, 16)] = zv16
            return carry
        lax.fori_loop(0, C2, zfa, 0)

        for h in range(4):
            for kk in range(TS // C2):
                pltpu.sync_copy(abufa,
                                out_sh.at[pl.ds(sid * TS + kk * C2, C2)])
            plsc.subcore_barrier()

            def schunk(g, carry):
                sbase = wid * (EPT // C2) + g * SB2
                ebase = sbase * C2
                pltpu.sync_copy(row2m_hbm.at[pl.ds(sbase, SB2)], rw2)
                pltpu.sync_copy(col2m_hbm.at[pl.ds(sbase, SB2)], cw2)
                pltpu.sync_copy(z_hbm.at[pl.ds(ebase, CB2)],
                                zwide.at[pl.ds(0, CB2)])
                gd = {0: pltpu.async_copy(q_hbm[h].at[rw2.at[0]],
                                          bufs[0], semg[0])}

                scat = [None, None]
                for j in range(SB2):
                    b = j % 2
                    gd[j].wait()
                    if j + 1 < SB2:
                        ob = (j + 1) % 2
                        if scat[ob] is not None:
                            scat[ob].wait()
                            scat[ob] = None
                        gd[j + 1] = pltpu.async_copy(
                            q_hbm[h].at[rw2.at[j + 1]], bufs[ob],
                            semg[ob])

                    buf = bufs[b]

                    def edge(e, carry2, _j=j, _buf=buf):
                        asc = zwide[pl.ds(_j * C2 + e, 16)][0]
                        av = jnp.full((16,), asc, F32)
                        for jj in range(NJQ):
                            sl = pl.ds(jj * 16, 16)
                            _buf[e, sl] = _buf[e, sl] * av
                        return carry2
                    lax.fori_loop(0, C2, edge, 0)

                    scat[b] = pltpu.async_copy(
                        buf, out_sh.at[cw2.at[j]], sems[b], add=True)
                for b in range(2):
                    if scat[b] is not None:
                        scat[b].wait()
                return carry
            lax.fori_loop(0, NSC2, schunk, 0)
            plsc.subcore_barrier()
            pltpu.sync_copy(out_sh.at[pl.ds(sid * TS, TS)],
                            out_hbm.at[h].at[cid].at[pl.ds(sid * TS, TS)])
            plsc.subcore_barrier()
            lax.fori_loop(0, C2, zfa, 0)

    return k(*xlq, z, row2m, col2m)


def kernel(x, edge_index, pos, W_emb, b_emb, Wl, bl, Wr, br, We, att, bias):
    row = edge_index[0]
    col = edge_index[1]
    posp = jnp.pad(pos, ((0, 0), (0, 13)))

    emb, s, xr, xl, q0, q1, q2, q3 = _tc_embed(x, W_emb, b_emb, Wl, bl,
                                               Wr, br)
    xlq = (q0, q1, q2, q3)

    eaE, lea_p, deg_p = _sc_edge_attr(posp, row.reshape(E // C0, C0),
                                      col.reshape(E // C0, C0))
    lea = lea_p[0, :N] + lea_p[1, :N]
    deg = deg_p[0, :N] + deg_p[1, :N]

    pad_i = jnp.zeros((E2P - E2,), I32)
    ar = jnp.arange(N, dtype=I32)
    row2 = jnp.concatenate([row, ar, pad_i])
    col2 = jnp.concatenate([col, ar, pad_i])
    ea2p8 = jnp.concatenate(
        [eaE.reshape(E * NG // 128, 128), lea.reshape(N * NG // 128, 128),
         jnp.zeros(((E2P - E2) // 8, 128), F32)], axis=0)
    inv = 1.0 / jnp.maximum(
        jnp.concatenate([jnp.ones((E,), F32), deg,
                         jnp.ones((E2P - E2,), F32)]), 1.0)
    scale8 = jnp.repeat(inv, NG).reshape(E2P // 8, 128)
    W_big = jnp.zeros((128, 8 * H), F32)
    for kq in range(8):
        W_big = lax.dynamic_update_slice(W_big, We, (kq * NG, kq * H))

    g2 = _tc_edge_embed(ea2p8, scale8, W_big)

    row2m = row2.reshape(E2P // C1, C1)
    col2m = col2.reshape(E2P // C1, C1)
    z, den_p = _sc_pass1(xl, xr, g2, row2m, col2m, att)
    outq = _sc_pass2(xlq, z, row2.reshape(E2P // C2, C2),
                     col2.reshape(E2P // C2, C2))

    s_out = _tc_finish(outq, den_p, s, bias)
    v = emb[:, :48].reshape(N, 16, 3)
    return (s_out, v)
